# Initial kernel scaffold; baseline (speedup 1.0000x reference)
import jax
import jax.numpy as jnp
from jax.experimental import pallas as pl


def kernel(x_r0, pos_r0, edge_index_r0, batch_r0, x_r1, pos_r1, edge_index_r1, batch_r1, x_p, pos_p, edge_index_p, batch_p, params):
    raise NotImplementedError("write your pallas kernel here")


# speedup vs baseline: 1.7899x; 1.7899x over previous
import functools

import jax
import jax.numpy as jnp
import numpy as np
from jax import lax
from jax.experimental import pallas as pl
from jax.experimental.pallas import tpu as pltpu
from jax.experimental.pallas import tpu_sc as plsc

N = 10000
E = 160000
NM = 3
N3 = N * NM
E3 = E * NM
NODE_FDIM = 128
NS = 16
DEMB = 32
NG = 16
MAX_RADIUS = 10.0
TD = 32

BE = 768
BN = 1000
ALPHA = 1.0 / np.sqrt(NS)
ALPHA2 = 1.0 / np.sqrt(2 * NS)
SQRT3 = np.sqrt(3.0)


def _mm(a, w, b):
    return jnp.dot(a, w, preferred_element_type=jnp.float32) + b


def _node_body(x_ref, pos_ref, w1_ref, b1_ref, w2_ref, b2_ref, t1_ref):
    h = _mm(jax.nn.relu(_mm(x_ref[...], w1_ref[...], b1_ref[...])),
            w2_ref[...], b2_ref[...])
    pad = jnp.zeros((BN, TD - NS - 3), jnp.float32)
    t1_ref[...] = jnp.concatenate([h, pos_ref[...], pad], axis=1)


def _node_mlp(x, pos, p):
    grid = (N3 // BN,)
    return pl.pallas_call(
        _node_body,
        grid=grid,
        in_specs=[
            pl.BlockSpec((BN, NODE_FDIM), lambda i: (i, 0)),
            pl.BlockSpec((BN, 3), lambda i: (i, 0)),
            pl.BlockSpec((NODE_FDIM, NS), lambda i: (0, 0)),
            pl.BlockSpec((1, NS), lambda i: (0, 0)),
            pl.BlockSpec((NS, NS), lambda i: (0, 0)),
            pl.BlockSpec((1, NS), lambda i: (0, 0)),
        ],
        out_specs=pl.BlockSpec((BN, TD), lambda i: (i, 0)),
        out_shape=jax.ShapeDtypeStruct((N3, TD), jnp.float32),
    )(x, pos, p["node_W1"], p["node_b1"].reshape(1, NS),
      p["node_W2"], p["node_b2"].reshape(1, NS))


def _edge1_body(gs_ref, gd_ref, ew1_ref, eb1_ref, ew2_ref, eb2_ref,
                fw1_ref, fb1_ref, fw2_ref, fb2_ref,
                tp_ref, ea_ref, sh_ref):
    gs = gs_ref[...]
    gd = gd_ref[...]
    ev = gd[:, NS:NS + 3] - gs[:, NS:NS + 3]
    d2 = (ev[:, 0:1] * ev[:, 0:1] + ev[:, 1:2] * ev[:, 1:2]
          + ev[:, 2:3] * ev[:, 2:3])
    dist = jnp.sqrt(d2 + 1e-12)
    sh1 = SQRT3 * ev / dist
    step = MAX_RADIUS / (DEMB - 1)
    mu = step * lax.broadcasted_iota(jnp.int32, (1, DEMB), 1).astype(jnp.float32)
    coeff = -0.5 / step ** 2
    rbf = jnp.exp(coeff * (dist - mu) ** 2)
    ea = _mm(jax.nn.relu(_mm(rbf, ew1_ref[...], eb1_ref[...])),
             ew2_ref[...], eb2_ref[...])
    e_in = jnp.concatenate([ea, gd[:, :NS], gs[:, :NS]], axis=1)
    w = _mm(jax.nn.relu(_mm(e_in, fw1_ref[...], fb1_ref[...])),
            fw2_ref[...], fb2_ref[...])
    u = gs[:, :NS]
    o0 = jnp.zeros((BE, NS), jnp.float32)
    t1 = jnp.zeros((BE, NS), jnp.float32)
    for i in range(NS):
        ui = u[:, i:i + 1]
        o0 = o0 + ui * w[:, i * NS:(i + 1) * NS]
        t1 = t1 + ui * w[:, NS * NS + i * NS:NS * NS + (i + 1) * NS]
    parts = [ALPHA * o0]
    for k in range(3):
        parts.append(ALPHA * t1 * sh1[:, k:k + 1])
    tp_ref[...] = jnp.concatenate(parts, axis=1)
    ea_ref[...] = ea
    sh_ref[...] = jnp.concatenate([sh1, jnp.zeros((BE, 1), jnp.float32)],
                                  axis=1)


def _edge1(gsrc, gdst, p):
    grid = (E3 // BE,)
    wspec = lambda s: pl.BlockSpec(s, lambda i: (0, 0))
    return pl.pallas_call(
        _edge1_body,
        grid=grid,
        in_specs=[
            pl.BlockSpec((BE, TD), lambda i: (i, 0)),
            pl.BlockSpec((BE, TD), lambda i: (i, 0)),
            wspec((DEMB, NS)), wspec((1, NS)), wspec((NS, NS)), wspec((1, NS)),
            wspec((3 * NS, 3 * NS)), wspec((1, 3 * NS)),
            wspec((3 * NS, 2 * NS * NS)), wspec((1, 2 * NS * NS)),
        ],
        out_specs=[
            pl.BlockSpec((BE, 4 * NS), lambda i: (i, 0)),
            pl.BlockSpec((BE, NS), lambda i: (i, 0)),
            pl.BlockSpec((BE, 4), lambda i: (i, 0)),
        ],
        out_shape=[
            jax.ShapeDtypeStruct((E3, 4 * NS), jnp.float32),
            jax.ShapeDtypeStruct((E3, NS), jnp.float32),
            jax.ShapeDtypeStruct((E3, 4), jnp.float32),
        ],
    )(gsrc, gdst, p["edge_W1"], p["edge_b1"].reshape(1, NS),
      p["edge_W2"], p["edge_b2"].reshape(1, NS),
      p["fc0_W1"], p["fc0_b1"].reshape(1, 3 * NS),
      p["fc0_W2"], p["fc0_b2"].reshape(1, 2 * NS * NS))


def _x1_body(t1_ref, pa_ref, pb_ref, cnt_ref, x1_ref, x1a_ref):
    h = t1_ref[:, :NS]
    agg = pa_ref[...] + pb_ref[...]
    rec = 1.0 / jnp.maximum(cnt_ref[...], 1.0)
    agg = agg * rec
    lo = h + agg[:, :NS]
    x1_ref[...] = jnp.concatenate([lo, agg[:, NS:]], axis=1)
    x1a_ref[...] = lo


def _x1_stage(t1, pa, pb, cnt):
    grid = (N3 // BN,)
    return pl.pallas_call(
        _x1_body,
        grid=grid,
        in_specs=[
            pl.BlockSpec((BN, TD), lambda i: (i, 0)),
            pl.BlockSpec((BN, 4 * NS), lambda i: (i, 0)),
            pl.BlockSpec((BN, 4 * NS), lambda i: (i, 0)),
            pl.BlockSpec((BN, 1), lambda i: (i, 0)),
        ],
        out_specs=[
            pl.BlockSpec((BN, 4 * NS), lambda i: (i, 0)),
            pl.BlockSpec((BN, NS), lambda i: (i, 0)),
        ],
        out_shape=[
            jax.ShapeDtypeStruct((N3, 4 * NS), jnp.float32),
            jax.ShapeDtypeStruct((N3, NS), jnp.float32),
        ],
    )(t1, pa, pb, cnt)


def _edge2_body(gs_ref, gd_ref, ea_ref, sh_ref, fw1_ref, fb1_ref,
                fw2_ref, fb2_ref, tp_ref):
    gs = gs_ref[...]
    ea = ea_ref[...]
    sh = sh_ref[...]
    e_in = jnp.concatenate([ea, gd_ref[...], gs[:, :NS]], axis=1)
    w = _mm(jax.nn.relu(_mm(e_in, fw1_ref[...], fb1_ref[...])),
            fw2_ref[...], fb2_ref[...])
    s0 = gs[:, :NS]
    pvec = jnp.zeros((BE, NS), jnp.float32)
    for k in range(3):
        pvec = pvec + gs[:, NS + k * NS:NS + (k + 1) * NS] * sh[:, k:k + 1]
    o0 = jnp.zeros((BE, NS), jnp.float32)
    o3 = jnp.zeros((BE, NS), jnp.float32)
    for i in range(NS):
        o0 = o0 + s0[:, i:i + 1] * w[:, i * NS:(i + 1) * NS]
        o3 = o3 + pvec[:, i:i + 1] * w[:, NS * NS + i * NS:NS * NS + (i + 1) * NS]
    tp_ref[...] = ALPHA2 * (o0 + o3 * (1.0 / SQRT3))


def _edge2(g2src, g2dst, ea, sh, p):
    fw2 = jnp.concatenate([p["fc1_W2"][:, 0:NS * NS],
                           p["fc1_W2"][:, 3 * NS * NS:4 * NS * NS]], axis=1)
    fb2 = jnp.concatenate([p["fc1_b2"][0:NS * NS],
                           p["fc1_b2"][3 * NS * NS:4 * NS * NS]]).reshape(1, -1)
    grid = (E3 // BE,)
    wspec = lambda s: pl.BlockSpec(s, lambda i: (0, 0))
    return pl.pallas_call(
        _edge2_body,
        grid=grid,
        in_specs=[
            pl.BlockSpec((BE, 4 * NS), lambda i: (i, 0)),
            pl.BlockSpec((BE, NS), lambda i: (i, 0)),
            pl.BlockSpec((BE, NS), lambda i: (i, 0)),
            pl.BlockSpec((BE, 4), lambda i: (i, 0)),
            wspec((3 * NS, 3 * NS)), wspec((1, 3 * NS)),
            wspec((3 * NS, 2 * NS * NS)), wspec((1, 2 * NS * NS)),
        ],
        out_specs=pl.BlockSpec((BE, NS), lambda i: (i, 0)),
        out_shape=jax.ShapeDtypeStruct((E3, NS), jnp.float32),
    )(g2src, g2dst, ea, sh, p["fc1_W1"], p["fc1_b1"].reshape(1, 3 * NS),
      fw2, fb2)


def _final_body(x1a_ref, qa_ref, qb_ref, cnt_ref, bat_ref,
                w1_ref, b1_ref, w2_ref, b2_ref, w3_ref, b3_ref, out_ref):
    rec = 1.0 / jnp.maximum(cnt_ref[...], 1.0)
    xf = x1a_ref[...] + (qa_ref[...] + qb_ref[...]) * rec
    hh = jax.nn.relu(_mm(xf, w1_ref[...], b1_ref[...]))
    hh = jax.nn.relu(_mm(hh, w2_ref[...], b2_ref[...]))
    sn = _mm(hh, w3_ref[...], b3_ref[...])
    gid = lax.broadcasted_iota(jnp.int32, (1, 3 * NG), 1).astype(jnp.float32)
    mask = (bat_ref[...] == gid).astype(jnp.float32)
    part = jnp.sum(mask * sn, axis=0, keepdims=True)

    @pl.when(pl.program_id(0) == 0)
    def _():
        out_ref[...] = jnp.zeros((1, 3 * NG), jnp.float32)

    out_ref[...] += part


def _final_stage(x1a, qa, qb, cnt, batf, p):
    grid = (N3 // BN,)
    wspec = lambda s: pl.BlockSpec(s, lambda i: (0, 0))
    return pl.pallas_call(
        _final_body,
        grid=grid,
        in_specs=[
            pl.BlockSpec((BN, NS), lambda i: (i, 0)),
            pl.BlockSpec((BN, NS), lambda i: (i, 0)),
            pl.BlockSpec((BN, NS), lambda i: (i, 0)),
            pl.BlockSpec((BN, 1), lambda i: (i, 0)),
            pl.BlockSpec((BN, 1), lambda i: (i, 0)),
            wspec((NS, 2 * NS)), wspec((1, 2 * NS)),
            wspec((2 * NS, NS)), wspec((1, NS)),
            wspec((NS, 1)), wspec((1, 1)),
        ],
        out_specs=pl.BlockSpec((1, 3 * NG), lambda i: (0, 0)),
        out_shape=jax.ShapeDtypeStruct((1, 3 * NG), jnp.float32),
    )(x1a, qa, qb, cnt, batf,
      p["sn_W1"], p["sn_b1"].reshape(1, 2 * NS),
      p["sn_W2"], p["sn_b2"].reshape(1, NS),
      p["sn_W3"], p["sn_b3"].reshape(1, 1))


def _gather_rows(table, idx):
    return table[idx]


def _scatter_mean_parts(vals, idx, with_counts):
    s = jax.ops.segment_sum(vals, idx, num_segments=N3)
    pa = s
    pb = jnp.zeros_like(s)
    cnt = None
    if with_counts:
        cnt = jax.ops.segment_sum(jnp.ones((vals.shape[0], 1), jnp.float32),
                                  idx, num_segments=N3)
    return pa, pb, cnt


def kernel(x_r0, pos_r0, edge_index_r0, batch_r0,
           x_r1, pos_r1, edge_index_r1, batch_r1,
           x_p, pos_p, edge_index_p, batch_p, params):
    p = params
    x_all = jnp.concatenate([x_r0, x_r1, x_p], axis=0)
    pos_all = jnp.concatenate([pos_r0, pos_r1, pos_p], axis=0)
    src_all = jnp.concatenate([edge_index_r0[0], edge_index_r1[0] + N,
                               edge_index_p[0] + 2 * N])
    dst_all = jnp.concatenate([edge_index_r0[1], edge_index_r1[1] + N,
                               edge_index_p[1] + 2 * N])
    batf = jnp.concatenate([batch_r0, batch_r1 + NG, batch_p + 2 * NG]
                           ).astype(jnp.float32).reshape(N3, 1)

    t1 = _node_mlp(x_all, pos_all, p)
    gsrc = _gather_rows(t1, src_all)
    gdst = _gather_rows(t1, dst_all)
    tp, ea, sh = _edge1(gsrc, gdst, p)
    pa, pb, cnt = _scatter_mean_parts(tp, dst_all, with_counts=True)
    x1, x1a = _x1_stage(t1, pa, pb, cnt)
    g2src = _gather_rows(x1, src_all)
    g2dst = _gather_rows(x1a, dst_all)
    tp2 = _edge2(g2src, g2dst, ea, sh, p)
    qa, qb, _ = _scatter_mean_parts(tp2, dst_all, with_counts=False)
    s48 = _final_stage(x1a, qa, qb, cnt, batf, p)[0]
    return (s48[2 * NG:3 * NG] - s48[0:NG] - s48[NG:2 * NG]).reshape(NG, 1)


# speedup vs baseline: 2.4680x; 1.3788x over previous
import functools

import jax
import jax.numpy as jnp
import numpy as np
from jax import lax
from jax.experimental import pallas as pl
from jax.experimental.pallas import tpu as pltpu
from jax.experimental.pallas import tpu_sc as plsc

N = 10000
E = 160000
NM = 3
N3 = N * NM
E3 = E * NM
N3P = 30720
E3P = 491520
NODE_FDIM = 128
NS = 16
DEMB = 32
NG = 16
MAX_RADIUS = 10.0
TD = 32

BE = 768
BN = 1024
ALPHA = 1.0 / np.sqrt(NS)
ALPHA2 = 1.0 / np.sqrt(2 * NS)
SQRT3 = np.sqrt(3.0)


def _mm(a, w, b):
    return jnp.dot(a, w, preferred_element_type=jnp.float32) + b


def _node_body(x_ref, pos_ref, w1_ref, b1_ref, w2_ref, b2_ref, t1_ref):
    h = _mm(jax.nn.relu(_mm(x_ref[...], w1_ref[...], b1_ref[...])),
            w2_ref[...], b2_ref[...])
    pad = jnp.zeros((BN, TD - NS - 3), jnp.float32)
    t1_ref[...] = jnp.concatenate([h, pos_ref[...], pad], axis=1)


def _node_mlp(x, pos, p):
    grid = (N3P // BN,)
    return pl.pallas_call(
        _node_body,
        grid=grid,
        in_specs=[
            pl.BlockSpec((BN, NODE_FDIM), lambda i: (i, 0)),
            pl.BlockSpec((BN, 3), lambda i: (i, 0)),
            pl.BlockSpec((NODE_FDIM, NS), lambda i: (0, 0)),
            pl.BlockSpec((1, NS), lambda i: (0, 0)),
            pl.BlockSpec((NS, NS), lambda i: (0, 0)),
            pl.BlockSpec((1, NS), lambda i: (0, 0)),
        ],
        out_specs=pl.BlockSpec((BN, TD), lambda i: (i, 0)),
        out_shape=jax.ShapeDtypeStruct((N3P, TD), jnp.float32),
    )(x, pos, p["node_W1"], p["node_b1"].reshape(1, NS),
      p["node_W2"], p["node_b2"].reshape(1, NS))


def _edge1_body(gs_ref, gd_ref, ew1_ref, eb1_ref, ew2_ref, eb2_ref,
                fw1_ref, fb1_ref, fw2_ref, fb2_ref,
                tpa_ref, tpb_ref, ea_ref, sh_ref):
    gs = gs_ref[...]
    gd = gd_ref[...]
    ev = gd[:, NS:NS + 3] - gs[:, NS:NS + 3]
    d2 = (ev[:, 0:1] * ev[:, 0:1] + ev[:, 1:2] * ev[:, 1:2]
          + ev[:, 2:3] * ev[:, 2:3])
    dist = jnp.sqrt(d2 + 1e-12)
    sh1 = SQRT3 * ev / dist
    step = MAX_RADIUS / (DEMB - 1)
    mu = step * lax.broadcasted_iota(jnp.int32, (1, DEMB), 1).astype(jnp.float32)
    coeff = -0.5 / step ** 2
    rbf = jnp.exp(coeff * (dist - mu) ** 2)
    ea = _mm(jax.nn.relu(_mm(rbf, ew1_ref[...], eb1_ref[...])),
             ew2_ref[...], eb2_ref[...])
    e_in = jnp.concatenate([ea, gd[:, :NS], gs[:, :NS]], axis=1)
    w = _mm(jax.nn.relu(_mm(e_in, fw1_ref[...], fb1_ref[...])),
            fw2_ref[...], fb2_ref[...])
    u = gs[:, :NS]
    o0 = jnp.zeros((BE, NS), jnp.float32)
    t1 = jnp.zeros((BE, NS), jnp.float32)
    for i in range(NS):
        ui = u[:, i:i + 1]
        o0 = o0 + ui * w[:, i * NS:(i + 1) * NS]
        t1 = t1 + ui * w[:, NS * NS + i * NS:NS * NS + (i + 1) * NS]
    tpa_ref[...] = jnp.concatenate(
        [ALPHA * o0, ALPHA * t1 * sh1[:, 0:1]], axis=1)
    tpb_ref[...] = jnp.concatenate(
        [ALPHA * t1 * sh1[:, 1:2], ALPHA * t1 * sh1[:, 2:3]], axis=1)
    ea_ref[...] = ea
    sh_ref[...] = jnp.concatenate([sh1, jnp.zeros((BE, 1), jnp.float32)],
                                  axis=1)


def _edge1(gsrc, gdst, p):
    grid = (E3P // BE,)
    wspec = lambda s: pl.BlockSpec(s, lambda i: (0, 0))
    return pl.pallas_call(
        _edge1_body,
        grid=grid,
        in_specs=[
            pl.BlockSpec((BE, TD), lambda i: (i, 0)),
            pl.BlockSpec((BE, TD), lambda i: (i, 0)),
            wspec((DEMB, NS)), wspec((1, NS)), wspec((NS, NS)), wspec((1, NS)),
            wspec((3 * NS, 3 * NS)), wspec((1, 3 * NS)),
            wspec((3 * NS, 2 * NS * NS)), wspec((1, 2 * NS * NS)),
        ],
        out_specs=[
            pl.BlockSpec((BE, 2 * NS), lambda i: (i, 0)),
            pl.BlockSpec((BE, 2 * NS), lambda i: (i, 0)),
            pl.BlockSpec((BE, NS), lambda i: (i, 0)),
            pl.BlockSpec((BE, 4), lambda i: (i, 0)),
        ],
        out_shape=[
            jax.ShapeDtypeStruct((E3P, 2 * NS), jnp.float32),
            jax.ShapeDtypeStruct((E3P, 2 * NS), jnp.float32),
            jax.ShapeDtypeStruct((E3P, NS), jnp.float32),
            jax.ShapeDtypeStruct((E3P, 4), jnp.float32),
        ],
    )(gsrc, gdst, p["edge_W1"], p["edge_b1"].reshape(1, NS),
      p["edge_W2"], p["edge_b2"].reshape(1, NS),
      p["fc0_W1"], p["fc0_b1"].reshape(1, 3 * NS),
      p["fc0_W2"], p["fc0_b2"].reshape(1, 2 * NS * NS))


def _x1_body(t1_ref, pa0_ref, pa1_ref, pb0_ref, pb1_ref, cnt_ref,
             lo_ref, hi_ref):
    h = t1_ref[:, :NS]
    rec = 1.0 / jnp.maximum(cnt_ref[...], 1.0)
    agg_a = (pa0_ref[...] + pa1_ref[...]) * rec
    agg_b = (pb0_ref[...] + pb1_ref[...]) * rec
    lo_ref[...] = jnp.concatenate([h + agg_a[:, :NS], agg_a[:, NS:]], axis=1)
    hi_ref[...] = agg_b


def _x1_stage(t1, pa0, pa1, pb0, pb1, cnt):
    grid = (N3P // BN,)
    return pl.pallas_call(
        _x1_body,
        grid=grid,
        in_specs=[
            pl.BlockSpec((BN, TD), lambda i: (i, 0)),
            pl.BlockSpec((BN, 2 * NS), lambda i: (i, 0)),
            pl.BlockSpec((BN, 2 * NS), lambda i: (i, 0)),
            pl.BlockSpec((BN, 2 * NS), lambda i: (i, 0)),
            pl.BlockSpec((BN, 2 * NS), lambda i: (i, 0)),
            pl.BlockSpec((BN, 1), lambda i: (i, 0)),
        ],
        out_specs=[
            pl.BlockSpec((BN, 2 * NS), lambda i: (i, 0)),
            pl.BlockSpec((BN, 2 * NS), lambda i: (i, 0)),
        ],
        out_shape=[
            jax.ShapeDtypeStruct((N3P, 2 * NS), jnp.float32),
            jax.ShapeDtypeStruct((N3P, 2 * NS), jnp.float32),
        ],
    )(t1, pa0, pa1, pb0, pb1, cnt)


def _edge2_body(ga_ref, gb_ref, gd_ref, ea_ref, sh_ref, fw1_ref, fb1_ref,
                fw2_ref, fb2_ref, tp_ref):
    ga = ga_ref[...]
    gb = gb_ref[...]
    ea = ea_ref[...]
    sh = sh_ref[...]
    e_in = jnp.concatenate([ea, gd_ref[:, :NS], ga[:, :NS]], axis=1)
    w = _mm(jax.nn.relu(_mm(e_in, fw1_ref[...], fb1_ref[...])),
            fw2_ref[...], fb2_ref[...])
    s0 = ga[:, :NS]
    s1k = [ga[:, NS:2 * NS], gb[:, :NS], gb[:, NS:2 * NS]]
    pvec = jnp.zeros((BE, NS), jnp.float32)
    for k in range(3):
        pvec = pvec + s1k[k] * sh[:, k:k + 1]
    o0 = jnp.zeros((BE, NS), jnp.float32)
    o3 = jnp.zeros((BE, NS), jnp.float32)
    for i in range(NS):
        o0 = o0 + s0[:, i:i + 1] * w[:, i * NS:(i + 1) * NS]
        o3 = o3 + pvec[:, i:i + 1] * w[:, NS * NS + i * NS:NS * NS + (i + 1) * NS]
    tp_ref[...] = ALPHA2 * (o0 + o3 * (1.0 / SQRT3))


def _edge2(ga, gb, gd, ea, sh, p):
    fw2 = jnp.concatenate([p["fc1_W2"][:, 0:NS * NS],
                           p["fc1_W2"][:, 3 * NS * NS:4 * NS * NS]], axis=1)
    fb2 = jnp.concatenate([p["fc1_b2"][0:NS * NS],
                           p["fc1_b2"][3 * NS * NS:4 * NS * NS]]).reshape(1, -1)
    grid = (E3P // BE,)
    wspec = lambda s: pl.BlockSpec(s, lambda i: (0, 0))
    return pl.pallas_call(
        _edge2_body,
        grid=grid,
        in_specs=[
            pl.BlockSpec((BE, 2 * NS), lambda i: (i, 0)),
            pl.BlockSpec((BE, 2 * NS), lambda i: (i, 0)),
            pl.BlockSpec((BE, 2 * NS), lambda i: (i, 0)),
            pl.BlockSpec((BE, NS), lambda i: (i, 0)),
            pl.BlockSpec((BE, 4), lambda i: (i, 0)),
            wspec((3 * NS, 3 * NS)), wspec((1, 3 * NS)),
            wspec((3 * NS, 2 * NS * NS)), wspec((1, 2 * NS * NS)),
        ],
        out_specs=pl.BlockSpec((BE, NS), lambda i: (i, 0)),
        out_shape=jax.ShapeDtypeStruct((E3P, NS), jnp.float32),
    )(ga, gb, gd, ea, sh, p["fc1_W1"], p["fc1_b1"].reshape(1, 3 * NS),
      fw2, fb2)


def _final_body(x1lo_ref, qa_ref, qb_ref, cnt_ref, bat_ref,
                w1_ref, b1_ref, w2_ref, b2_ref, w3_ref, b3_ref, out_ref):
    rec = 1.0 / jnp.maximum(cnt_ref[...], 1.0)
    xf = x1lo_ref[:, :NS] + (qa_ref[...] + qb_ref[...]) * rec
    hh = jax.nn.relu(_mm(xf, w1_ref[...], b1_ref[...]))
    hh = jax.nn.relu(_mm(hh, w2_ref[...], b2_ref[...]))
    sn = _mm(hh, w3_ref[...], b3_ref[...])
    gid = lax.broadcasted_iota(jnp.int32, (1, 3 * NG), 1).astype(jnp.float32)
    mask = (bat_ref[...] == gid).astype(jnp.float32)
    part = jnp.sum(mask * sn, axis=0, keepdims=True)

    @pl.when(pl.program_id(0) == 0)
    def _():
        out_ref[...] = jnp.zeros((1, 3 * NG), jnp.float32)

    out_ref[...] += part


def _final_stage(x1lo, qa, qb, cnt, batf, p):
    grid = (N3P // BN,)
    wspec = lambda s: pl.BlockSpec(s, lambda i: (0, 0))
    return pl.pallas_call(
        _final_body,
        grid=grid,
        in_specs=[
            pl.BlockSpec((BN, 2 * NS), lambda i: (i, 0)),
            pl.BlockSpec((BN, NS), lambda i: (i, 0)),
            pl.BlockSpec((BN, NS), lambda i: (i, 0)),
            pl.BlockSpec((BN, 1), lambda i: (i, 0)),
            pl.BlockSpec((BN, 1), lambda i: (i, 0)),
            wspec((NS, 2 * NS)), wspec((1, 2 * NS)),
            wspec((2 * NS, NS)), wspec((1, NS)),
            wspec((NS, 1)), wspec((1, 1)),
        ],
        out_specs=pl.BlockSpec((1, 3 * NG), lambda i: (0, 0)),
        out_shape=jax.ShapeDtypeStruct((1, 3 * NG), jnp.float32),
    )(x1lo, qa, qb, cnt, batf,
      p["sn_W1"], p["sn_b1"].reshape(1, 2 * NS),
      p["sn_W2"], p["sn_b2"].reshape(1, NS),
      p["sn_W3"], p["sn_b3"].reshape(1, 1))


NW = 32
PW = E3P // NW
SUB = 120
NSTR = 8
CH = NSTR * SUB
NCH = PW // CH
NIR = E3P // SUB
STRIPE = N3P // 16
_MESH = plsc.VectorSubcoreMesh(core_axis_name="c", subcore_axis_name="s")
_SC_PARAMS = pltpu.CompilerParams(use_tc_tiling_on_sc=False)


def _sc_gather1(table, srci, dsti):
    @functools.partial(
        pl.kernel,
        out_type=[jax.ShapeDtypeStruct((E3P, TD), jnp.float32),
                  jax.ShapeDtypeStruct((E3P, TD), jnp.float32)],
        mesh=_MESH,
        compiler_params=_SC_PARAMS,
        scratch_types=[pltpu.VMEM((NSTR, SUB), jnp.int32),
                       pltpu.VMEM((NSTR, SUB), jnp.int32),
                       pltpu.VMEM((CH, TD), jnp.float32),
                       pltpu.VMEM((CH, TD), jnp.float32),
                       pltpu.VMEM_SHARED((N3P, TD), jnp.float32),
                       pltpu.SemaphoreType.DMA],
    )
    def k(t_hbm, is_hbm, id_hbm, outs, outd, ia_v, ib_v, rowsa, rowsb,
          tab, sem):
        cid = lax.axis_index("c")
        sid = lax.axis_index("s")
        pltpu.sync_copy(t_hbm.at[pl.ds(sid * STRIPE, STRIPE)],
                        tab.at[pl.ds(sid * STRIPE, STRIPE)])
        plsc.subcore_barrier()
        wid = cid * (NW // 2) + sid

        def body(g, carry):
            r0 = wid * (PW // SUB) + g * NSTR
            pltpu.sync_copy(is_hbm.at[pl.ds(r0, NSTR)], ia_v)
            pltpu.sync_copy(id_hbm.at[pl.ds(r0, NSTR)], ib_v)
            cps = []
            for j in range(NSTR):
                cps.append(pltpu.async_copy(
                    tab.at[ia_v.at[j]], rowsa.at[pl.ds(j * SUB, SUB)], sem))
                cps.append(pltpu.async_copy(
                    tab.at[ib_v.at[j]], rowsb.at[pl.ds(j * SUB, SUB)], sem))
            for cp in cps:
                cp.wait()
            e0 = wid * PW + g * CH
            pltpu.sync_copy(rowsa, outs.at[pl.ds(e0, CH)])
            pltpu.sync_copy(rowsb, outd.at[pl.ds(e0, CH)])
            return carry

        lax.fori_loop(0, NCH, body, 0)

    return k(table, srci, dsti)


def _sc_gather2(x1lo, x1hi, srci, dsti):
    @functools.partial(
        pl.kernel,
        out_type=[jax.ShapeDtypeStruct((E3P, TD), jnp.float32),
                  jax.ShapeDtypeStruct((E3P, TD), jnp.float32),
                  jax.ShapeDtypeStruct((E3P, TD), jnp.float32)],
        mesh=_MESH,
        compiler_params=_SC_PARAMS,
        scratch_types=[pltpu.VMEM((NSTR, SUB), jnp.int32),
                       pltpu.VMEM((NSTR, SUB), jnp.int32),
                       pltpu.VMEM((CH, TD), jnp.float32),
                       pltpu.VMEM((CH, TD), jnp.float32),
                       pltpu.VMEM_SHARED((N3P, TD), jnp.float32),
                       pltpu.SemaphoreType.DMA],
    )
    def k(lo_hbm, hi_hbm, is_hbm, id_hbm, outa, outd, outb,
          ia_v, ib_v, rowsa, rowsb, tab, sem):
        cid = lax.axis_index("c")
        sid = lax.axis_index("s")
        wid = cid * (NW // 2) + sid
        pltpu.sync_copy(lo_hbm.at[pl.ds(sid * STRIPE, STRIPE)],
                        tab.at[pl.ds(sid * STRIPE, STRIPE)])
        plsc.subcore_barrier()

        def body_lo(g, carry):
            r0 = wid * (PW // SUB) + g * NSTR
            pltpu.sync_copy(is_hbm.at[pl.ds(r0, NSTR)], ia_v)
            pltpu.sync_copy(id_hbm.at[pl.ds(r0, NSTR)], ib_v)
            cps = []
            for j in range(NSTR):
                cps.append(pltpu.async_copy(
                    tab.at[ia_v.at[j]], rowsa.at[pl.ds(j * SUB, SUB)], sem))
                cps.append(pltpu.async_copy(
                    tab.at[ib_v.at[j]], rowsb.at[pl.ds(j * SUB, SUB)], sem))
            for cp in cps:
                cp.wait()
            e0 = wid * PW + g * CH
            pltpu.sync_copy(rowsa, outa.at[pl.ds(e0, CH)])
            pltpu.sync_copy(rowsb, outd.at[pl.ds(e0, CH)])
            return carry

        lax.fori_loop(0, NCH, body_lo, 0)
        plsc.subcore_barrier()
        pltpu.sync_copy(hi_hbm.at[pl.ds(sid * STRIPE, STRIPE)],
                        tab.at[pl.ds(sid * STRIPE, STRIPE)])
        plsc.subcore_barrier()

        def body_hi(g, carry):
            r0 = wid * (PW // SUB) + g * NSTR
            pltpu.sync_copy(is_hbm.at[pl.ds(r0, NSTR)], ia_v)
            cps = []
            for j in range(NSTR):
                cps.append(pltpu.async_copy(
                    tab.at[ia_v.at[j]], rowsa.at[pl.ds(j * SUB, SUB)], sem))
            for cp in cps:
                cp.wait()
            pltpu.sync_copy(rowsa, outb.at[pl.ds(wid * PW + g * CH, CH)])
            return carry

        lax.fori_loop(0, NCH, body_hi, 0)

    return k(x1lo, x1hi, srci, dsti)


def _sc_scatter1(tpa, tpb, idx2, zrows, zcnt, ones):
    @functools.partial(
        pl.kernel,
        out_type=[jax.ShapeDtypeStruct((2 * N3P, TD), jnp.float32),
                  jax.ShapeDtypeStruct((2 * N3P, TD), jnp.float32),
                  jax.ShapeDtypeStruct((2 * N3P,), jnp.float32)],
        mesh=_MESH,
        compiler_params=_SC_PARAMS,
        scratch_types=[pltpu.VMEM((NSTR, SUB), jnp.int32),
                       pltpu.VMEM((CH, TD), jnp.float32),
                       pltpu.VMEM((SUB,), jnp.float32),
                       pltpu.VMEM_SHARED((N3P, TD), jnp.float32),
                       pltpu.VMEM_SHARED((N3P,), jnp.float32)],
    )
    def k(va_hbm, vb_hbm, i_hbm, zr_hbm, zc_hbm, on_hbm,
          parta, partb, cntp, idx_v, vals_v, ones_v, acc, cacc):
        cid = lax.axis_index("c")
        sid = lax.axis_index("s")
        wid = cid * (NW // 2) + sid
        pltpu.sync_copy(zr_hbm, acc.at[pl.ds(sid * STRIPE, STRIPE)])
        pltpu.sync_copy(zc_hbm, cacc.at[pl.ds(sid * STRIPE, STRIPE)])
        pltpu.sync_copy(on_hbm, ones_v)
        plsc.subcore_barrier()

        def body_a(g, carry):
            r0 = wid * (PW // SUB) + g * NSTR
            pltpu.sync_copy(i_hbm.at[pl.ds(r0, NSTR)], idx_v)
            pltpu.sync_copy(va_hbm.at[pl.ds(wid * PW + g * CH, CH)], vals_v)
            for j in range(NSTR):
                pltpu.sync_copy(vals_v.at[pl.ds(j * SUB, SUB)],
                                acc.at[idx_v.at[j]], add=True)
                pltpu.sync_copy(ones_v, cacc.at[idx_v.at[j]], add=True)
            return carry

        lax.fori_loop(0, NCH, body_a, 0)
        plsc.subcore_barrier()
        pltpu.sync_copy(acc.at[pl.ds(sid * STRIPE, STRIPE)],
                        parta.at[pl.ds(cid * N3P + sid * STRIPE, STRIPE)])
        pltpu.sync_copy(cacc.at[pl.ds(sid * STRIPE, STRIPE)],
                        cntp.at[pl.ds(cid * N3P + sid * STRIPE, STRIPE)])
        pltpu.sync_copy(zr_hbm, acc.at[pl.ds(sid * STRIPE, STRIPE)])
        plsc.subcore_barrier()

        def body_b(g, carry):
            r0 = wid * (PW // SUB) + g * NSTR
            pltpu.sync_copy(i_hbm.at[pl.ds(r0, NSTR)], idx_v)
            pltpu.sync_copy(vb_hbm.at[pl.ds(wid * PW + g * CH, CH)], vals_v)
            for j in range(NSTR):
                pltpu.sync_copy(vals_v.at[pl.ds(j * SUB, SUB)],
                                acc.at[idx_v.at[j]], add=True)
            return carry

        lax.fori_loop(0, NCH, body_b, 0)
        plsc.subcore_barrier()
        pltpu.sync_copy(acc.at[pl.ds(sid * STRIPE, STRIPE)],
                        partb.at[pl.ds(cid * N3P + sid * STRIPE, STRIPE)])

    return k(tpa, tpb, idx2, zrows, zcnt, ones)


def _sc_scatter2(tp2, idx2, zrows16):
    @functools.partial(
        pl.kernel,
        out_type=jax.ShapeDtypeStruct((2 * N3P, NS), jnp.float32),
        mesh=_MESH,
        compiler_params=_SC_PARAMS,
        scratch_types=[pltpu.VMEM((NSTR, SUB), jnp.int32),
                       pltpu.VMEM((CH, NS), jnp.float32),
                       pltpu.VMEM_SHARED((N3P, NS), jnp.float32)],
    )
    def k(v_hbm, i_hbm, zr_hbm, part, idx_v, vals_v, acc):
        cid = lax.axis_index("c")
        sid = lax.axis_index("s")
        wid = cid * (NW // 2) + sid
        pltpu.sync_copy(zr_hbm, acc.at[pl.ds(sid * STRIPE, STRIPE)])
        plsc.subcore_barrier()

        def body(g, carry):
            r0 = wid * (PW // SUB) + g * NSTR
            pltpu.sync_copy(i_hbm.at[pl.ds(r0, NSTR)], idx_v)
            pltpu.sync_copy(v_hbm.at[pl.ds(wid * PW + g * CH, CH)], vals_v)
            for j in range(NSTR):
                pltpu.sync_copy(vals_v.at[pl.ds(j * SUB, SUB)],
                                acc.at[idx_v.at[j]], add=True)
            return carry

        lax.fori_loop(0, NCH, body, 0)
        plsc.subcore_barrier()
        pltpu.sync_copy(acc.at[pl.ds(sid * STRIPE, STRIPE)],
                        part.at[pl.ds(cid * N3P + sid * STRIPE, STRIPE)])

    return k(tp2, idx2, zrows16)


def kernel(x_r0, pos_r0, edge_index_r0, batch_r0,
           x_r1, pos_r1, edge_index_r1, batch_r1,
           x_p, pos_p, edge_index_p, batch_p, params):
    p = params
    npad = N3P - N3
    epad = E3P - E3
    x_all = jnp.concatenate(
        [x_r0, x_r1, x_p, jnp.zeros((npad, NODE_FDIM), jnp.float32)], axis=0)
    pos_all = jnp.concatenate(
        [pos_r0, pos_r1, pos_p, jnp.zeros((npad, 3), jnp.float32)], axis=0)
    src_all = jnp.concatenate([edge_index_r0[0], edge_index_r1[0] + N,
                               edge_index_p[0] + 2 * N,
                               jnp.zeros((epad,), jnp.int32)])
    dst_all = jnp.concatenate([edge_index_r0[1], edge_index_r1[1] + N,
                               edge_index_p[1] + 2 * N,
                               jnp.full((epad,), N3, jnp.int32)])
    batf = jnp.concatenate(
        [batch_r0, batch_r1 + NG, batch_p + 2 * NG,
         jnp.full((npad,), -1, jnp.int32)]).astype(jnp.float32).reshape(N3P, 1)
    src2 = src_all.reshape(NIR, SUB)
    dst2 = dst_all.reshape(NIR, SUB)

    t1 = _node_mlp(x_all, pos_all, p)
    gsrc, gdst = _sc_gather1(t1, src2, dst2)
    tpa, tpb, ea, sh = _edge1(gsrc, gdst, p)
    zrows = jnp.zeros((STRIPE, TD), jnp.float32)
    zcnt = jnp.zeros((STRIPE,), jnp.float32)
    ones = jnp.ones((SUB,), jnp.float32)
    parta, partb, cntp = _sc_scatter1(tpa, tpb, dst2, zrows, zcnt, ones)
    cnt = (cntp[:N3P] + cntp[N3P:]).reshape(N3P, 1)
    x1lo, x1hi = _x1_stage(t1, parta[:N3P], parta[N3P:],
                           partb[:N3P], partb[N3P:], cnt)
    ga, gd, gb = _sc_gather2(x1lo, x1hi, src2, dst2)
    tp2 = _edge2(ga, gb, gd, ea, sh, p)
    zrows16 = jnp.zeros((STRIPE, NS), jnp.float32)
    part2 = _sc_scatter2(tp2, dst2, zrows16)
    s48 = _final_stage(x1lo, part2[:N3P], part2[N3P:], cnt, batf, p)[0]
    return (s48[2 * NG:3 * NG] - s48[0:NG] - s48[NG:2 * NG]).reshape(NG, 1)


# speedup vs baseline: 7.2778x; 2.9489x over previous
import functools

import jax
import jax.numpy as jnp
import numpy as np
from jax import lax
from jax.experimental import pallas as pl
from jax.experimental.pallas import tpu as pltpu
from jax.experimental.pallas import tpu_sc as plsc

N = 10000
E = 160000
NM = 3
N3 = N * NM
E3 = E * NM
N3P = 30720
E3P = 491520
NODE_FDIM = 128
NS = 16
DEMB = 32
NG = 16
MAX_RADIUS = 10.0
TD = 32

BE = 3840
BN = 1024
ALPHA = 1.0 / np.sqrt(NS)
ALPHA2 = 1.0 / np.sqrt(2 * NS)
SQRT3 = np.sqrt(3.0)


def _mm(a, w, b):
    return jnp.dot(a, w, preferred_element_type=jnp.float32) + b


def _mmb(a, w, b):
    return jnp.dot(a.astype(jnp.bfloat16), w,
                   preferred_element_type=jnp.float32) + b


def _mk_routes():
    r1 = np.zeros((NS, 2 * NS * NS), np.float32)
    s1 = np.zeros((2 * NS * NS, 2 * NS), np.float32)
    r2 = np.zeros((2 * NS, 2 * NS * NS), np.float32)
    s2 = np.zeros((2 * NS * NS, NS), np.float32)
    for i in range(NS):
        for o in range(NS):
            for j in range(2):
                r1[i, j * NS * NS + i * NS + o] = 1.0
                s1[j * NS * NS + i * NS + o, j * NS + o] = 1.0
            r2[i, i * NS + o] = 1.0
            r2[NS + i, NS * NS + i * NS + o] = 1.0
            s2[i * NS + o, o] = 1.0
            s2[NS * NS + i * NS + o, o] = 1.0 / SQRT3
    return r1, s1, r2, s2


_R1_np, _S1_np, _R2_np, _S2_np = _mk_routes()


def _node_body(x_ref, pos_ref, w1_ref, b1_ref, w2_ref, b2_ref, t1_ref):
    h = _mm(jax.nn.relu(_mm(x_ref[...], w1_ref[...], b1_ref[...])),
            w2_ref[...], b2_ref[...])
    pad = jnp.zeros((BN, TD - NS - 3), jnp.float32)
    t1_ref[...] = jnp.concatenate([h, pos_ref[...], pad], axis=1)


def _node_mlp(x, pos, p):
    grid = (N3P // BN,)
    return pl.pallas_call(
        _node_body,
        grid=grid,
        in_specs=[
            pl.BlockSpec((BN, NODE_FDIM), lambda i: (i, 0)),
            pl.BlockSpec((BN, 3), lambda i: (i, 0)),
            pl.BlockSpec((NODE_FDIM, NS), lambda i: (0, 0)),
            pl.BlockSpec((1, NS), lambda i: (0, 0)),
            pl.BlockSpec((NS, NS), lambda i: (0, 0)),
            pl.BlockSpec((1, NS), lambda i: (0, 0)),
        ],
        out_specs=pl.BlockSpec((BN, TD), lambda i: (i, 0)),
        out_shape=jax.ShapeDtypeStruct((N3P, TD), jnp.float32),
    )(x, pos, p["node_W1"], p["node_b1"].reshape(1, NS),
      p["node_W2"], p["node_b2"].reshape(1, NS))


def _edge1_body(gs_ref, gd_ref, ew1_ref, eb1_ref, ew2_ref, eb2_ref,
                fw1_ref, fb1_ref, fw2_ref, fb2_ref, r_ref, s_ref,
                tpa_ref, tpb_ref, ea_ref, sh_ref):
    gs = gs_ref[...]
    gd = gd_ref[...]
    ev = gd[:, NS:NS + 3] - gs[:, NS:NS + 3]
    d2 = (ev[:, 0:1] * ev[:, 0:1] + ev[:, 1:2] * ev[:, 1:2]
          + ev[:, 2:3] * ev[:, 2:3])
    dist = jnp.sqrt(d2 + 1e-12)
    sh1 = SQRT3 * ev / dist
    step = MAX_RADIUS / (DEMB - 1)
    mu = step * lax.broadcasted_iota(jnp.int32, (1, DEMB), 1).astype(jnp.float32)
    coeff = -0.5 / step ** 2
    rbf = jnp.exp(coeff * (dist - mu) ** 2)
    ea = _mm(jax.nn.relu(_mm(rbf, ew1_ref[...], eb1_ref[...])),
             ew2_ref[...], eb2_ref[...])
    e_in = jnp.concatenate([ea, gd[:, :NS], gs[:, :NS]], axis=1)
    w = _mmb(jax.nn.relu(_mmb(e_in, fw1_ref[...], fb1_ref[...])),
             fw2_ref[...], fb2_ref[...])
    u = gs[:, :NS]
    rep = jnp.dot(u.astype(jnp.bfloat16), r_ref[...],
                  preferred_element_type=jnp.float32)
    prod = rep * w
    ot = jnp.dot(prod.astype(jnp.bfloat16), s_ref[...],
                 preferred_element_type=jnp.float32)
    o0 = ot[:, :NS]
    t1 = ot[:, NS:]
    tpa_ref[...] = jnp.concatenate(
        [ALPHA * o0, ALPHA * t1 * sh1[:, 0:1]], axis=1)
    tpb_ref[...] = jnp.concatenate(
        [ALPHA * t1 * sh1[:, 1:2], ALPHA * t1 * sh1[:, 2:3]], axis=1)
    ea_ref[...] = ea
    sh_ref[...] = jnp.concatenate([sh1, jnp.zeros((BE, 1), jnp.float32)],
                                  axis=1)


def _edge1(gsrc, gdst, p):
    grid = (E3P // BE,)
    wspec = lambda s: pl.BlockSpec(s, lambda i: (0, 0))
    return pl.pallas_call(
        _edge1_body,
        grid=grid,
        in_specs=[
            pl.BlockSpec((BE, TD), lambda i: (i, 0)),
            pl.BlockSpec((BE, TD), lambda i: (i, 0)),
            wspec((DEMB, NS)), wspec((1, NS)), wspec((NS, NS)), wspec((1, NS)),
            wspec((3 * NS, 3 * NS)), wspec((1, 3 * NS)),
            wspec((3 * NS, 2 * NS * NS)), wspec((1, 2 * NS * NS)),
            wspec((NS, 2 * NS * NS)), wspec((2 * NS * NS, 2 * NS)),
        ],
        out_specs=[
            pl.BlockSpec((BE, 2 * NS), lambda i: (i, 0)),
            pl.BlockSpec((BE, 2 * NS), lambda i: (i, 0)),
            pl.BlockSpec((BE, NS), lambda i: (i, 0)),
            pl.BlockSpec((BE, 4), lambda i: (i, 0)),
        ],
        out_shape=[
            jax.ShapeDtypeStruct((E3P, 2 * NS), jnp.float32),
            jax.ShapeDtypeStruct((E3P, 2 * NS), jnp.float32),
            jax.ShapeDtypeStruct((E3P, NS), jnp.float32),
            jax.ShapeDtypeStruct((E3P, 4), jnp.float32),
        ],
    )(gsrc, gdst, p["edge_W1"], p["edge_b1"].reshape(1, NS),
      p["edge_W2"], p["edge_b2"].reshape(1, NS),
      p["fc0_W1"].astype(jnp.bfloat16), p["fc0_b1"].reshape(1, 3 * NS),
      p["fc0_W2"].astype(jnp.bfloat16), p["fc0_b2"].reshape(1, 2 * NS * NS),
      jnp.asarray(_R1_np).astype(jnp.bfloat16),
      jnp.asarray(_S1_np).astype(jnp.bfloat16))


def _x1_body(t1_ref, pa0_ref, pa1_ref, pb0_ref, pb1_ref, cnt_ref,
             lo_ref, hi_ref):
    h = t1_ref[:, :NS]
    rec = 1.0 / jnp.maximum(cnt_ref[...], 1.0)
    agg_a = (pa0_ref[...] + pa1_ref[...]) * rec
    agg_b = (pb0_ref[...] + pb1_ref[...]) * rec
    lo_ref[...] = jnp.concatenate([h + agg_a[:, :NS], agg_a[:, NS:]], axis=1)
    hi_ref[...] = agg_b


def _x1_stage(t1, pa0, pa1, pb0, pb1, cnt):
    grid = (N3P // BN,)
    return pl.pallas_call(
        _x1_body,
        grid=grid,
        in_specs=[
            pl.BlockSpec((BN, TD), lambda i: (i, 0)),
            pl.BlockSpec((BN, 2 * NS), lambda i: (i, 0)),
            pl.BlockSpec((BN, 2 * NS), lambda i: (i, 0)),
            pl.BlockSpec((BN, 2 * NS), lambda i: (i, 0)),
            pl.BlockSpec((BN, 2 * NS), lambda i: (i, 0)),
            pl.BlockSpec((BN, 1), lambda i: (i, 0)),
        ],
        out_specs=[
            pl.BlockSpec((BN, 2 * NS), lambda i: (i, 0)),
            pl.BlockSpec((BN, 2 * NS), lambda i: (i, 0)),
        ],
        out_shape=[
            jax.ShapeDtypeStruct((N3P, 2 * NS), jnp.float32),
            jax.ShapeDtypeStruct((N3P, 2 * NS), jnp.float32),
        ],
    )(t1, pa0, pa1, pb0, pb1, cnt)


def _edge2_body(ga_ref, gb_ref, gd_ref, ea_ref, sh_ref, fw1_ref, fb1_ref,
                fw2_ref, fb2_ref, r_ref, s_ref, tp_ref):
    ga = ga_ref[...]
    gb = gb_ref[...]
    ea = ea_ref[...]
    sh = sh_ref[...]
    e_in = jnp.concatenate([ea, gd_ref[:, :NS], ga[:, :NS]], axis=1)
    w = _mmb(jax.nn.relu(_mmb(e_in, fw1_ref[...], fb1_ref[...])),
             fw2_ref[...], fb2_ref[...])
    s0 = ga[:, :NS]
    s1k = [ga[:, NS:2 * NS], gb[:, :NS], gb[:, NS:2 * NS]]
    pvec = jnp.zeros((BE, NS), jnp.float32)
    for k in range(3):
        pvec = pvec + s1k[k] * sh[:, k:k + 1]
    cat = jnp.concatenate([s0, pvec], axis=1)
    rep = jnp.dot(cat.astype(jnp.bfloat16), r_ref[...],
                  preferred_element_type=jnp.float32)
    prod = rep * w
    o0 = jnp.dot(prod.astype(jnp.bfloat16), s_ref[...],
                 preferred_element_type=jnp.float32)
    tp_ref[...] = ALPHA2 * o0


def _edge2(ga, gb, gd, ea, sh, p):
    fw2 = jnp.concatenate([p["fc1_W2"][:, 0:NS * NS],
                           p["fc1_W2"][:, 3 * NS * NS:4 * NS * NS]], axis=1)
    fb2 = jnp.concatenate([p["fc1_b2"][0:NS * NS],
                           p["fc1_b2"][3 * NS * NS:4 * NS * NS]]).reshape(1, -1)
    grid = (E3P // BE,)
    wspec = lambda s: pl.BlockSpec(s, lambda i: (0, 0))
    return pl.pallas_call(
        _edge2_body,
        grid=grid,
        in_specs=[
            pl.BlockSpec((BE, 2 * NS), lambda i: (i, 0)),
            pl.BlockSpec((BE, 2 * NS), lambda i: (i, 0)),
            pl.BlockSpec((BE, 2 * NS), lambda i: (i, 0)),
            pl.BlockSpec((BE, NS), lambda i: (i, 0)),
            pl.BlockSpec((BE, 4), lambda i: (i, 0)),
            wspec((3 * NS, 3 * NS)), wspec((1, 3 * NS)),
            wspec((3 * NS, 2 * NS * NS)), wspec((1, 2 * NS * NS)),
            wspec((2 * NS, 2 * NS * NS)), wspec((2 * NS * NS, NS)),
        ],
        out_specs=pl.BlockSpec((BE, NS), lambda i: (i, 0)),
        out_shape=jax.ShapeDtypeStruct((E3P, NS), jnp.float32),
    )(ga, gb, gd, ea, sh,
      p["fc1_W1"].astype(jnp.bfloat16), p["fc1_b1"].reshape(1, 3 * NS),
      fw2.astype(jnp.bfloat16), fb2,
      jnp.asarray(_R2_np).astype(jnp.bfloat16),
      jnp.asarray(_S2_np).astype(jnp.bfloat16))


def _final_body(x1lo_ref, qa_ref, qb_ref, cnt_ref, bat_ref,
                w1_ref, b1_ref, w2_ref, b2_ref, w3_ref, b3_ref, out_ref):
    rec = 1.0 / jnp.maximum(cnt_ref[...], 1.0)
    xf = x1lo_ref[:, :NS] + (qa_ref[...] + qb_ref[...]) * rec
    hh = jax.nn.relu(_mm(xf, w1_ref[...], b1_ref[...]))
    hh = jax.nn.relu(_mm(hh, w2_ref[...], b2_ref[...]))
    sn = _mm(hh, w3_ref[...], b3_ref[...])
    gid = lax.broadcasted_iota(jnp.int32, (1, 3 * NG), 1).astype(jnp.float32)
    mask = (bat_ref[...] == gid).astype(jnp.float32)
    part = jnp.sum(mask * sn, axis=0, keepdims=True)

    @pl.when(pl.program_id(0) == 0)
    def _():
        out_ref[...] = jnp.zeros((1, 3 * NG), jnp.float32)

    out_ref[...] += part


def _final_stage(x1lo, qa, qb, cnt, batf, p):
    grid = (N3P // BN,)
    wspec = lambda s: pl.BlockSpec(s, lambda i: (0, 0))
    return pl.pallas_call(
        _final_body,
        grid=grid,
        in_specs=[
            pl.BlockSpec((BN, 2 * NS), lambda i: (i, 0)),
            pl.BlockSpec((BN, NS), lambda i: (i, 0)),
            pl.BlockSpec((BN, NS), lambda i: (i, 0)),
            pl.BlockSpec((BN, 1), lambda i: (i, 0)),
            pl.BlockSpec((BN, 1), lambda i: (i, 0)),
            wspec((NS, 2 * NS)), wspec((1, 2 * NS)),
            wspec((2 * NS, NS)), wspec((1, NS)),
            wspec((NS, 1)), wspec((1, 1)),
        ],
        out_specs=pl.BlockSpec((1, 3 * NG), lambda i: (0, 0)),
        out_shape=jax.ShapeDtypeStruct((1, 3 * NG), jnp.float32),
    )(x1lo, qa, qb, cnt, batf,
      p["sn_W1"], p["sn_b1"].reshape(1, 2 * NS),
      p["sn_W2"], p["sn_b2"].reshape(1, NS),
      p["sn_W3"], p["sn_b3"].reshape(1, 1))


NW = 32
PW = E3P // NW
SUB = 120
NSTR = 8
CH = NSTR * SUB
NCH = PW // CH
NIR = E3P // SUB
STRIPE = N3P // 16
_SC_PARAMS = pltpu.CompilerParams(use_tc_tiling_on_sc=False)


def _sc_mesh():
    return plsc.VectorSubcoreMesh(core_axis_name="c", subcore_axis_name="s")


def _sc_gather1(table, srci, dsti):
    @functools.partial(
        pl.kernel,
        out_type=[jax.ShapeDtypeStruct((E3P, TD), jnp.float32),
                  jax.ShapeDtypeStruct((E3P, TD), jnp.float32)],
        mesh=_sc_mesh(),
        compiler_params=_SC_PARAMS,
        scratch_types=[pltpu.VMEM((NSTR, SUB), jnp.int32),
                       pltpu.VMEM((NSTR, SUB), jnp.int32),
                       pltpu.VMEM((CH, TD), jnp.float32),
                       pltpu.VMEM((CH, TD), jnp.float32),
                       pltpu.VMEM_SHARED((N3P, TD), jnp.float32),
                       pltpu.SemaphoreType.DMA],
    )
    def k(t_hbm, is_hbm, id_hbm, outs, outd, ia_v, ib_v, rowsa, rowsb,
          tab, sem):
        cid = lax.axis_index("c")
        sid = lax.axis_index("s")
        pltpu.sync_copy(t_hbm.at[pl.ds(sid * STRIPE, STRIPE)],
                        tab.at[pl.ds(sid * STRIPE, STRIPE)])
        plsc.subcore_barrier()
        wid = cid * (NW // 2) + sid

        def body(g, carry):
            r0 = wid * (PW // SUB) + g * NSTR
            pltpu.sync_copy(is_hbm.at[pl.ds(r0, NSTR)], ia_v)
            pltpu.sync_copy(id_hbm.at[pl.ds(r0, NSTR)], ib_v)
            cps = []
            for j in range(NSTR):
                cps.append(pltpu.async_copy(
                    tab.at[ia_v.at[j]], rowsa.at[pl.ds(j * SUB, SUB)], sem))
                cps.append(pltpu.async_copy(
                    tab.at[ib_v.at[j]], rowsb.at[pl.ds(j * SUB, SUB)], sem))
            for cp in cps:
                cp.wait()
            e0 = wid * PW + g * CH
            pltpu.sync_copy(rowsa, outs.at[pl.ds(e0, CH)])
            pltpu.sync_copy(rowsb, outd.at[pl.ds(e0, CH)])
            return carry

        lax.fori_loop(0, NCH, body, 0)

    return k(table, srci, dsti)


def _sc_gather2(x1lo, x1hi, srci, dsti):
    @functools.partial(
        pl.kernel,
        out_type=[jax.ShapeDtypeStruct((E3P, TD), jnp.float32),
                  jax.ShapeDtypeStruct((E3P, TD), jnp.float32),
                  jax.ShapeDtypeStruct((E3P, TD), jnp.float32)],
        mesh=_sc_mesh(),
        compiler_params=_SC_PARAMS,
        scratch_types=[pltpu.VMEM((NSTR, SUB), jnp.int32),
                       pltpu.VMEM((NSTR, SUB), jnp.int32),
                       pltpu.VMEM((CH, TD), jnp.float32),
                       pltpu.VMEM((CH, TD), jnp.float32),
                       pltpu.VMEM_SHARED((N3P, TD), jnp.float32),
                       pltpu.SemaphoreType.DMA],
    )
    def k(lo_hbm, hi_hbm, is_hbm, id_hbm, outa, outd, outb,
          ia_v, ib_v, rowsa, rowsb, tab, sem):
        cid = lax.axis_index("c")
        sid = lax.axis_index("s")
        wid = cid * (NW // 2) + sid
        pltpu.sync_copy(lo_hbm.at[pl.ds(sid * STRIPE, STRIPE)],
                        tab.at[pl.ds(sid * STRIPE, STRIPE)])
        plsc.subcore_barrier()

        def body_lo(g, carry):
            r0 = wid * (PW // SUB) + g * NSTR
            pltpu.sync_copy(is_hbm.at[pl.ds(r0, NSTR)], ia_v)
            pltpu.sync_copy(id_hbm.at[pl.ds(r0, NSTR)], ib_v)
            cps = []
            for j in range(NSTR):
                cps.append(pltpu.async_copy(
                    tab.at[ia_v.at[j]], rowsa.at[pl.ds(j * SUB, SUB)], sem))
                cps.append(pltpu.async_copy(
                    tab.at[ib_v.at[j]], rowsb.at[pl.ds(j * SUB, SUB)], sem))
            for cp in cps:
                cp.wait()
            e0 = wid * PW + g * CH
            pltpu.sync_copy(rowsa, outa.at[pl.ds(e0, CH)])
            pltpu.sync_copy(rowsb, outd.at[pl.ds(e0, CH)])
            return carry

        lax.fori_loop(0, NCH, body_lo, 0)
        plsc.subcore_barrier()
        pltpu.sync_copy(hi_hbm.at[pl.ds(sid * STRIPE, STRIPE)],
                        tab.at[pl.ds(sid * STRIPE, STRIPE)])
        plsc.subcore_barrier()

        def body_hi(g, carry):
            r0 = wid * (PW // SUB) + g * NSTR
            pltpu.sync_copy(is_hbm.at[pl.ds(r0, NSTR)], ia_v)
            cps = []
            for j in range(NSTR):
                cps.append(pltpu.async_copy(
                    tab.at[ia_v.at[j]], rowsa.at[pl.ds(j * SUB, SUB)], sem))
            for cp in cps:
                cp.wait()
            pltpu.sync_copy(rowsa, outb.at[pl.ds(wid * PW + g * CH, CH)])
            return carry

        lax.fori_loop(0, NCH, body_hi, 0)

    return k(x1lo, x1hi, srci, dsti)


def _sc_scatter1(tpa, tpb, idx2, zrows, zcnt, ones):
    @functools.partial(
        pl.kernel,
        out_type=[jax.ShapeDtypeStruct((2 * N3P, TD), jnp.float32),
                  jax.ShapeDtypeStruct((2 * N3P, TD), jnp.float32),
                  jax.ShapeDtypeStruct((2 * N3P,), jnp.float32)],
        mesh=_sc_mesh(),
        compiler_params=_SC_PARAMS,
        scratch_types=[pltpu.VMEM((NSTR, SUB), jnp.int32),
                       pltpu.VMEM((CH, TD), jnp.float32),
                       pltpu.VMEM((SUB,), jnp.float32),
                       pltpu.VMEM_SHARED((N3P, TD), jnp.float32),
                       pltpu.VMEM_SHARED((N3P,), jnp.float32)],
    )
    def k(va_hbm, vb_hbm, i_hbm, zr_hbm, zc_hbm, on_hbm,
          parta, partb, cntp, idx_v, vals_v, ones_v, acc, cacc):
        cid = lax.axis_index("c")
        sid = lax.axis_index("s")
        wid = cid * (NW // 2) + sid
        pltpu.sync_copy(zr_hbm, acc.at[pl.ds(sid * STRIPE, STRIPE)])
        pltpu.sync_copy(zc_hbm, cacc.at[pl.ds(sid * STRIPE, STRIPE)])
        pltpu.sync_copy(on_hbm, ones_v)
        plsc.subcore_barrier()

        def body_a(g, carry):
            r0 = wid * (PW // SUB) + g * NSTR
            pltpu.sync_copy(i_hbm.at[pl.ds(r0, NSTR)], idx_v)
            pltpu.sync_copy(va_hbm.at[pl.ds(wid * PW + g * CH, CH)], vals_v)
            for j in range(NSTR):
                pltpu.sync_copy(vals_v.at[pl.ds(j * SUB, SUB)],
                                acc.at[idx_v.at[j]], add=True)
                pltpu.sync_copy(ones_v, cacc.at[idx_v.at[j]], add=True)
            return carry

        lax.fori_loop(0, NCH, body_a, 0)
        plsc.subcore_barrier()
        pltpu.sync_copy(acc.at[pl.ds(sid * STRIPE, STRIPE)],
                        parta.at[pl.ds(cid * N3P + sid * STRIPE, STRIPE)])
        pltpu.sync_copy(cacc.at[pl.ds(sid * STRIPE, STRIPE)],
                        cntp.at[pl.ds(cid * N3P + sid * STRIPE, STRIPE)])
        pltpu.sync_copy(zr_hbm, acc.at[pl.ds(sid * STRIPE, STRIPE)])
        plsc.subcore_barrier()

        def body_b(g, carry):
            r0 = wid * (PW // SUB) + g * NSTR
            pltpu.sync_copy(i_hbm.at[pl.ds(r0, NSTR)], idx_v)
            pltpu.sync_copy(vb_hbm.at[pl.ds(wid * PW + g * CH, CH)], vals_v)
            for j in range(NSTR):
                pltpu.sync_copy(vals_v.at[pl.ds(j * SUB, SUB)],
                                acc.at[idx_v.at[j]], add=True)
            return carry

        lax.fori_loop(0, NCH, body_b, 0)
        plsc.subcore_barrier()
        pltpu.sync_copy(acc.at[pl.ds(sid * STRIPE, STRIPE)],
                        partb.at[pl.ds(cid * N3P + sid * STRIPE, STRIPE)])

    return k(tpa, tpb, idx2, zrows, zcnt, ones)


def _sc_scatter2(tp2, idx2, zrows16):
    @functools.partial(
        pl.kernel,
        out_type=jax.ShapeDtypeStruct((2 * N3P, NS), jnp.float32),
        mesh=_sc_mesh(),
        compiler_params=_SC_PARAMS,
        scratch_types=[pltpu.VMEM((NSTR, SUB), jnp.int32),
                       pltpu.VMEM((CH, NS), jnp.float32),
                       pltpu.VMEM_SHARED((N3P, NS), jnp.float32)],
    )
    def k(v_hbm, i_hbm, zr_hbm, part, idx_v, vals_v, acc):
        cid = lax.axis_index("c")
        sid = lax.axis_index("s")
        wid = cid * (NW // 2) + sid
        pltpu.sync_copy(zr_hbm, acc.at[pl.ds(sid * STRIPE, STRIPE)])
        plsc.subcore_barrier()

        def body(g, carry):
            r0 = wid * (PW // SUB) + g * NSTR
            pltpu.sync_copy(i_hbm.at[pl.ds(r0, NSTR)], idx_v)
            pltpu.sync_copy(v_hbm.at[pl.ds(wid * PW + g * CH, CH)], vals_v)
            for j in range(NSTR):
                pltpu.sync_copy(vals_v.at[pl.ds(j * SUB, SUB)],
                                acc.at[idx_v.at[j]], add=True)
            return carry

        lax.fori_loop(0, NCH, body, 0)
        plsc.subcore_barrier()
        pltpu.sync_copy(acc.at[pl.ds(sid * STRIPE, STRIPE)],
                        part.at[pl.ds(cid * N3P + sid * STRIPE, STRIPE)])

    return k(tp2, idx2, zrows16)


def kernel(x_r0, pos_r0, edge_index_r0, batch_r0,
           x_r1, pos_r1, edge_index_r1, batch_r1,
           x_p, pos_p, edge_index_p, batch_p, params):
    p = params
    npad = N3P - N3
    epad = E3P - E3
    x_all = jnp.concatenate(
        [x_r0, x_r1, x_p, jnp.zeros((npad, NODE_FDIM), jnp.float32)], axis=0)
    pos_all = jnp.concatenate(
        [pos_r0, pos_r1, pos_p, jnp.zeros((npad, 3), jnp.float32)], axis=0)
    src_all = jnp.concatenate([edge_index_r0[0], edge_index_r1[0] + N,
                               edge_index_p[0] + 2 * N,
                               jnp.zeros((epad,), jnp.int32)])
    dst_all = jnp.concatenate([edge_index_r0[1], edge_index_r1[1] + N,
                               edge_index_p[1] + 2 * N,
                               jnp.full((epad,), N3, jnp.int32)])
    batf = jnp.concatenate(
        [batch_r0, batch_r1 + NG, batch_p + 2 * NG,
         jnp.full((npad,), -1, jnp.int32)]).astype(jnp.float32).reshape(N3P, 1)
    src2 = src_all.reshape(NIR, SUB)
    dst2 = dst_all.reshape(NIR, SUB)

    t1 = _node_mlp(x_all, pos_all, p)
    gsrc, gdst = _sc_gather1(t1, src2, dst2)
    tpa, tpb, ea, sh = _edge1(gsrc, gdst, p)
    zrows = jnp.zeros((STRIPE, TD), jnp.float32)
    zcnt = jnp.zeros((STRIPE,), jnp.float32)
    ones = jnp.ones((SUB,), jnp.float32)
    parta, partb, cntp = _sc_scatter1(tpa, tpb, dst2, zrows, zcnt, ones)
    cnt = (cntp[:N3P] + cntp[N3P:]).reshape(N3P, 1)
    x1lo, x1hi = _x1_stage(t1, parta[:N3P], parta[N3P:],
                           partb[:N3P], partb[N3P:], cnt)
    ga, gd, gb = _sc_gather2(x1lo, x1hi, src2, dst2)
    tp2 = _edge2(ga, gb, gd, ea, sh, p)
    zrows16 = jnp.zeros((STRIPE, NS), jnp.float32)
    part2 = _sc_scatter2(tp2, dst2, zrows16)
    s48 = _final_stage(x1lo, part2[:N3P], part2[N3P:], cnt, batf, p)[0]
    return (s48[2 * NG:3 * NG] - s48[0:NG] - s48[NG:2 * NG]).reshape(NG, 1)


# speedup vs baseline: 7.3020x; 1.0033x over previous
import functools

import jax
import jax.numpy as jnp
import numpy as np
from jax import lax
from jax.experimental import pallas as pl
from jax.experimental.pallas import tpu as pltpu
from jax.experimental.pallas import tpu_sc as plsc

N = 10000
E = 160000
NM = 3
N3 = N * NM
E3 = E * NM
N3P = 30720
E3P = 491520
NODE_FDIM = 128
NS = 16
DEMB = 32
NG = 16
MAX_RADIUS = 10.0
TD = 32

BE = 2560
BN = 1024
ALPHA = 1.0 / np.sqrt(NS)
ALPHA2 = 1.0 / np.sqrt(2 * NS)
SQRT3 = np.sqrt(3.0)


def _mm(a, w, b):
    return jnp.dot(a, w, preferred_element_type=jnp.float32) + b


def _mmb(a, w, b):
    return jnp.dot(a.astype(jnp.bfloat16), w,
                   preferred_element_type=jnp.float32) + b


def _mk_routes():
    r1 = np.zeros((NS, 2 * NS * NS), np.float32)
    s1 = np.zeros((2 * NS * NS, 2 * NS), np.float32)
    r2 = np.zeros((2 * NS, 2 * NS * NS), np.float32)
    s2 = np.zeros((2 * NS * NS, NS), np.float32)
    for i in range(NS):
        for o in range(NS):
            for j in range(2):
                r1[i, j * NS * NS + i * NS + o] = 1.0
                s1[j * NS * NS + i * NS + o, j * NS + o] = 1.0
            r2[i, i * NS + o] = 1.0
            r2[NS + i, NS * NS + i * NS + o] = 1.0
            s2[i * NS + o, o] = 1.0
            s2[NS * NS + i * NS + o, o] = 1.0 / SQRT3
    return r1, s1, r2, s2


_R1_np, _S1_np, _R2_np, _S2_np = _mk_routes()


def _node_body(x_ref, pos_ref, w1_ref, b1_ref, w2_ref, b2_ref, t1_ref):
    h = _mm(jax.nn.relu(_mm(x_ref[...], w1_ref[...], b1_ref[...])),
            w2_ref[...], b2_ref[...])
    pad = jnp.zeros((BN, TD - NS - 3), jnp.float32)
    t1_ref[...] = jnp.concatenate([h, pos_ref[...], pad], axis=1)


def _node_mlp(x, pos, p):
    grid = (N3P // BN,)
    return pl.pallas_call(
        _node_body,
        grid=grid,
        in_specs=[
            pl.BlockSpec((BN, NODE_FDIM), lambda i: (i, 0)),
            pl.BlockSpec((BN, 3), lambda i: (i, 0)),
            pl.BlockSpec((NODE_FDIM, NS), lambda i: (0, 0)),
            pl.BlockSpec((1, NS), lambda i: (0, 0)),
            pl.BlockSpec((NS, NS), lambda i: (0, 0)),
            pl.BlockSpec((1, NS), lambda i: (0, 0)),
        ],
        out_specs=pl.BlockSpec((BN, TD), lambda i: (i, 0)),
        out_shape=jax.ShapeDtypeStruct((N3P, TD), jnp.float32),
    )(x, pos, p["node_W1"], p["node_b1"].reshape(1, NS),
      p["node_W2"], p["node_b2"].reshape(1, NS))


def _edge1_body(gs_ref, gd_ref, ew1_ref, eb1_ref, ew2_ref, eb2_ref,
                fw1_ref, fb1_ref, fw2_ref, fb2_ref, r_ref, s_ref,
                tpa_ref, tpb_ref, ea_ref, sh_ref):
    gs = gs_ref[...]
    gd = gd_ref[...]
    ev = gd[:, NS:NS + 3] - gs[:, NS:NS + 3]
    d2 = (ev[:, 0:1] * ev[:, 0:1] + ev[:, 1:2] * ev[:, 1:2]
          + ev[:, 2:3] * ev[:, 2:3])
    dist = jnp.sqrt(d2 + 1e-12)
    sh1 = SQRT3 * ev / dist
    step = MAX_RADIUS / (DEMB - 1)
    mu = step * lax.broadcasted_iota(jnp.int32, (1, DEMB), 1).astype(jnp.float32)
    coeff = -0.5 / step ** 2
    rbf = jnp.exp(coeff * (dist - mu) ** 2)
    ea = _mm(jax.nn.relu(_mm(rbf, ew1_ref[...], eb1_ref[...])),
             ew2_ref[...], eb2_ref[...])
    e_in = jnp.concatenate([ea, gd[:, :NS], gs[:, :NS]], axis=1)
    w = _mmb(jax.nn.relu(_mmb(e_in, fw1_ref[...], fb1_ref[...])),
             fw2_ref[...], fb2_ref[...])
    u = gs[:, :NS]
    rep = jnp.dot(u.astype(jnp.bfloat16), r_ref[...],
                  preferred_element_type=jnp.float32)
    prod = rep * w
    ot = jnp.dot(prod.astype(jnp.bfloat16), s_ref[...],
                 preferred_element_type=jnp.float32)
    o0 = ot[:, :NS]
    t1 = ot[:, NS:]
    tpa_ref[...] = jnp.concatenate(
        [ALPHA * o0, ALPHA * t1 * sh1[:, 0:1]], axis=1)
    tpb_ref[...] = jnp.concatenate(
        [ALPHA * t1 * sh1[:, 1:2], ALPHA * t1 * sh1[:, 2:3]], axis=1)
    ea_ref[...] = ea
    sh_ref[...] = jnp.concatenate([sh1, jnp.zeros((BE, 1), jnp.float32)],
                                  axis=1)


def _edge1(gsrc, gdst, p):
    grid = (E3P // BE,)
    wspec = lambda s: pl.BlockSpec(s, lambda i: (0, 0))
    return pl.pallas_call(
        _edge1_body,
        grid=grid,
        in_specs=[
            pl.BlockSpec((BE, TD), lambda i: (i, 0)),
            pl.BlockSpec((BE, TD), lambda i: (i, 0)),
            wspec((DEMB, NS)), wspec((1, NS)), wspec((NS, NS)), wspec((1, NS)),
            wspec((3 * NS, 3 * NS)), wspec((1, 3 * NS)),
            wspec((3 * NS, 2 * NS * NS)), wspec((1, 2 * NS * NS)),
            wspec((NS, 2 * NS * NS)), wspec((2 * NS * NS, 2 * NS)),
        ],
        out_specs=[
            pl.BlockSpec((BE, 2 * NS), lambda i: (i, 0)),
            pl.BlockSpec((BE, 2 * NS), lambda i: (i, 0)),
            pl.BlockSpec((BE, NS), lambda i: (i, 0)),
            pl.BlockSpec((BE, 4), lambda i: (i, 0)),
        ],
        out_shape=[
            jax.ShapeDtypeStruct((E3P, 2 * NS), jnp.float32),
            jax.ShapeDtypeStruct((E3P, 2 * NS), jnp.float32),
            jax.ShapeDtypeStruct((E3P, NS), jnp.float32),
            jax.ShapeDtypeStruct((E3P, 4), jnp.float32),
        ],
    )(gsrc, gdst, p["edge_W1"], p["edge_b1"].reshape(1, NS),
      p["edge_W2"], p["edge_b2"].reshape(1, NS),
      p["fc0_W1"].astype(jnp.bfloat16), p["fc0_b1"].reshape(1, 3 * NS),
      p["fc0_W2"].astype(jnp.bfloat16), p["fc0_b2"].reshape(1, 2 * NS * NS),
      jnp.asarray(_R1_np).astype(jnp.bfloat16),
      jnp.asarray(_S1_np).astype(jnp.bfloat16))


def _x1_body(t1_ref, pa0_ref, pa1_ref, pb0_ref, pb1_ref, cnt_ref,
             lo_ref, hi_ref):
    h = t1_ref[:, :NS]
    rec = 1.0 / jnp.maximum(cnt_ref[...], 1.0)
    agg_a = (pa0_ref[...] + pa1_ref[...]) * rec
    agg_b = (pb0_ref[...] + pb1_ref[...]) * rec
    lo_ref[...] = jnp.concatenate([h + agg_a[:, :NS], agg_a[:, NS:]], axis=1)
    hi_ref[...] = agg_b


def _x1_stage(t1, pa0, pa1, pb0, pb1, cnt):
    grid = (N3P // BN,)
    return pl.pallas_call(
        _x1_body,
        grid=grid,
        in_specs=[
            pl.BlockSpec((BN, TD), lambda i: (i, 0)),
            pl.BlockSpec((BN, 2 * NS), lambda i: (i, 0)),
            pl.BlockSpec((BN, 2 * NS), lambda i: (i, 0)),
            pl.BlockSpec((BN, 2 * NS), lambda i: (i, 0)),
            pl.BlockSpec((BN, 2 * NS), lambda i: (i, 0)),
            pl.BlockSpec((BN, 1), lambda i: (i, 0)),
        ],
        out_specs=[
            pl.BlockSpec((BN, 2 * NS), lambda i: (i, 0)),
            pl.BlockSpec((BN, 2 * NS), lambda i: (i, 0)),
        ],
        out_shape=[
            jax.ShapeDtypeStruct((N3P, 2 * NS), jnp.float32),
            jax.ShapeDtypeStruct((N3P, 2 * NS), jnp.float32),
        ],
    )(t1, pa0, pa1, pb0, pb1, cnt)


def _edge2_body(ga_ref, gb_ref, gd_ref, ea_ref, sh_ref, fw1_ref, fb1_ref,
                fw2_ref, fb2_ref, r_ref, s_ref, tp_ref):
    ga = ga_ref[...]
    gb = gb_ref[...]
    ea = ea_ref[...]
    sh = sh_ref[...]
    e_in = jnp.concatenate([ea, gd_ref[:, :NS], ga[:, :NS]], axis=1)
    w = _mmb(jax.nn.relu(_mmb(e_in, fw1_ref[...], fb1_ref[...])),
             fw2_ref[...], fb2_ref[...])
    s0 = ga[:, :NS]
    s1k = [ga[:, NS:2 * NS], gb[:, :NS], gb[:, NS:2 * NS]]
    pvec = jnp.zeros((BE, NS), jnp.float32)
    for k in range(3):
        pvec = pvec + s1k[k] * sh[:, k:k + 1]
    cat = jnp.concatenate([s0, pvec], axis=1)
    rep = jnp.dot(cat.astype(jnp.bfloat16), r_ref[...],
                  preferred_element_type=jnp.float32)
    prod = rep * w
    o0 = jnp.dot(prod.astype(jnp.bfloat16), s_ref[...],
                 preferred_element_type=jnp.float32)
    tp_ref[...] = ALPHA2 * o0


def _edge2(ga, gb, gd, ea, sh, p):
    fw2 = jnp.concatenate([p["fc1_W2"][:, 0:NS * NS],
                           p["fc1_W2"][:, 3 * NS * NS:4 * NS * NS]], axis=1)
    fb2 = jnp.concatenate([p["fc1_b2"][0:NS * NS],
                           p["fc1_b2"][3 * NS * NS:4 * NS * NS]]).reshape(1, -1)
    grid = (E3P // BE,)
    wspec = lambda s: pl.BlockSpec(s, lambda i: (0, 0))
    return pl.pallas_call(
        _edge2_body,
        grid=grid,
        in_specs=[
            pl.BlockSpec((BE, 2 * NS), lambda i: (i, 0)),
            pl.BlockSpec((BE, 2 * NS), lambda i: (i, 0)),
            pl.BlockSpec((BE, 2 * NS), lambda i: (i, 0)),
            pl.BlockSpec((BE, NS), lambda i: (i, 0)),
            pl.BlockSpec((BE, 4), lambda i: (i, 0)),
            wspec((3 * NS, 3 * NS)), wspec((1, 3 * NS)),
            wspec((3 * NS, 2 * NS * NS)), wspec((1, 2 * NS * NS)),
            wspec((2 * NS, 2 * NS * NS)), wspec((2 * NS * NS, NS)),
        ],
        out_specs=pl.BlockSpec((BE, NS), lambda i: (i, 0)),
        out_shape=jax.ShapeDtypeStruct((E3P, NS), jnp.float32),
    )(ga, gb, gd, ea, sh,
      p["fc1_W1"].astype(jnp.bfloat16), p["fc1_b1"].reshape(1, 3 * NS),
      fw2.astype(jnp.bfloat16), fb2,
      jnp.asarray(_R2_np).astype(jnp.bfloat16),
      jnp.asarray(_S2_np).astype(jnp.bfloat16))


def _final_body(x1lo_ref, qa_ref, qb_ref, cnt_ref, bat_ref,
                w1_ref, b1_ref, w2_ref, b2_ref, w3_ref, b3_ref, out_ref):
    rec = 1.0 / jnp.maximum(cnt_ref[...], 1.0)
    xf = x1lo_ref[:, :NS] + (qa_ref[...] + qb_ref[...]) * rec
    hh = jax.nn.relu(_mm(xf, w1_ref[...], b1_ref[...]))
    hh = jax.nn.relu(_mm(hh, w2_ref[...], b2_ref[...]))
    sn = _mm(hh, w3_ref[...], b3_ref[...])
    gid = lax.broadcasted_iota(jnp.int32, (1, 3 * NG), 1).astype(jnp.float32)
    mask = (bat_ref[...] == gid).astype(jnp.float32)
    part = jnp.sum(mask * sn, axis=0, keepdims=True)

    @pl.when(pl.program_id(0) == 0)
    def _():
        out_ref[...] = jnp.zeros((1, 3 * NG), jnp.float32)

    out_ref[...] += part


def _final_stage(x1lo, qa, qb, cnt, batf, p):
    grid = (N3P // BN,)
    wspec = lambda s: pl.BlockSpec(s, lambda i: (0, 0))
    return pl.pallas_call(
        _final_body,
        grid=grid,
        in_specs=[
            pl.BlockSpec((BN, 2 * NS), lambda i: (i, 0)),
            pl.BlockSpec((BN, NS), lambda i: (i, 0)),
            pl.BlockSpec((BN, NS), lambda i: (i, 0)),
            pl.BlockSpec((BN, 1), lambda i: (i, 0)),
            pl.BlockSpec((BN, 1), lambda i: (i, 0)),
            wspec((NS, 2 * NS)), wspec((1, 2 * NS)),
            wspec((2 * NS, NS)), wspec((1, NS)),
            wspec((NS, 1)), wspec((1, 1)),
        ],
        out_specs=pl.BlockSpec((1, 3 * NG), lambda i: (0, 0)),
        out_shape=jax.ShapeDtypeStruct((1, 3 * NG), jnp.float32),
    )(x1lo, qa, qb, cnt, batf,
      p["sn_W1"], p["sn_b1"].reshape(1, 2 * NS),
      p["sn_W2"], p["sn_b2"].reshape(1, NS),
      p["sn_W3"], p["sn_b3"].reshape(1, 1))


NW = 32
PW = E3P // NW
SUB = 120
NSTR = 8
CH = NSTR * SUB
NCH = PW // CH
NIR = E3P // SUB
STRIPE = N3P // 16
_SC_PARAMS = pltpu.CompilerParams(use_tc_tiling_on_sc=False)


def _sc_mesh():
    return plsc.VectorSubcoreMesh(core_axis_name="c", subcore_axis_name="s")


def _sc_gather1(table, srci, dsti):
    @functools.partial(
        pl.kernel,
        out_type=[jax.ShapeDtypeStruct((E3P, TD), jnp.float32),
                  jax.ShapeDtypeStruct((E3P, TD), jnp.float32)],
        mesh=_sc_mesh(),
        compiler_params=_SC_PARAMS,
        scratch_types=[pltpu.VMEM((NSTR, SUB), jnp.int32),
                       pltpu.VMEM((NSTR, SUB), jnp.int32),
                       pltpu.VMEM((CH, TD), jnp.float32),
                       pltpu.VMEM((CH, TD), jnp.float32),
                       pltpu.VMEM_SHARED((N3P, TD), jnp.float32),
                       pltpu.SemaphoreType.DMA],
    )
    def k(t_hbm, is_hbm, id_hbm, outs, outd, ia_v, ib_v, rowsa, rowsb,
          tab, sem):
        cid = lax.axis_index("c")
        sid = lax.axis_index("s")
        pltpu.sync_copy(t_hbm.at[pl.ds(sid * STRIPE, STRIPE)],
                        tab.at[pl.ds(sid * STRIPE, STRIPE)])
        plsc.subcore_barrier()
        wid = cid * (NW // 2) + sid

        def body(g, carry):
            r0 = wid * (PW // SUB) + g * NSTR
            pltpu.sync_copy(is_hbm.at[pl.ds(r0, NSTR)], ia_v)
            pltpu.sync_copy(id_hbm.at[pl.ds(r0, NSTR)], ib_v)
            cps = []
            for j in range(NSTR):
                cps.append(pltpu.async_copy(
                    tab.at[ia_v.at[j]], rowsa.at[pl.ds(j * SUB, SUB)], sem))
                cps.append(pltpu.async_copy(
                    tab.at[ib_v.at[j]], rowsb.at[pl.ds(j * SUB, SUB)], sem))
            for cp in cps:
                cp.wait()
            e0 = wid * PW + g * CH
            pltpu.sync_copy(rowsa, outs.at[pl.ds(e0, CH)])
            pltpu.sync_copy(rowsb, outd.at[pl.ds(e0, CH)])
            return carry

        lax.fori_loop(0, NCH, body, 0)

    return k(table, srci, dsti)


def _sc_gather2(x1lo, x1hi, srci, dsti):
    @functools.partial(
        pl.kernel,
        out_type=[jax.ShapeDtypeStruct((E3P, TD), jnp.float32),
                  jax.ShapeDtypeStruct((E3P, TD), jnp.float32),
                  jax.ShapeDtypeStruct((E3P, TD), jnp.float32)],
        mesh=_sc_mesh(),
        compiler_params=_SC_PARAMS,
        scratch_types=[pltpu.VMEM((NSTR, SUB), jnp.int32),
                       pltpu.VMEM((NSTR, SUB), jnp.int32),
                       pltpu.VMEM((CH, TD), jnp.float32),
                       pltpu.VMEM((CH, TD), jnp.float32),
                       pltpu.VMEM_SHARED((N3P, TD), jnp.float32),
                       pltpu.SemaphoreType.DMA],
    )
    def k(lo_hbm, hi_hbm, is_hbm, id_hbm, outa, outd, outb,
          ia_v, ib_v, rowsa, rowsb, tab, sem):
        cid = lax.axis_index("c")
        sid = lax.axis_index("s")
        wid = cid * (NW // 2) + sid
        pltpu.sync_copy(lo_hbm.at[pl.ds(sid * STRIPE, STRIPE)],
                        tab.at[pl.ds(sid * STRIPE, STRIPE)])
        plsc.subcore_barrier()

        def body_lo(g, carry):
            r0 = wid * (PW // SUB) + g * NSTR
            pltpu.sync_copy(is_hbm.at[pl.ds(r0, NSTR)], ia_v)
            pltpu.sync_copy(id_hbm.at[pl.ds(r0, NSTR)], ib_v)
            cps = []
            for j in range(NSTR):
                cps.append(pltpu.async_copy(
                    tab.at[ia_v.at[j]], rowsa.at[pl.ds(j * SUB, SUB)], sem))
                cps.append(pltpu.async_copy(
                    tab.at[ib_v.at[j]], rowsb.at[pl.ds(j * SUB, SUB)], sem))
            for cp in cps:
                cp.wait()
            e0 = wid * PW + g * CH
            pltpu.sync_copy(rowsa, outa.at[pl.ds(e0, CH)])
            pltpu.sync_copy(rowsb, outd.at[pl.ds(e0, CH)])
            return carry

        lax.fori_loop(0, NCH, body_lo, 0)
        plsc.subcore_barrier()
        pltpu.sync_copy(hi_hbm.at[pl.ds(sid * STRIPE, STRIPE)],
                        tab.at[pl.ds(sid * STRIPE, STRIPE)])
        plsc.subcore_barrier()

        def body_hi(g, carry):
            r0 = wid * (PW // SUB) + g * NSTR
            pltpu.sync_copy(is_hbm.at[pl.ds(r0, NSTR)], ia_v)
            cps = []
            for j in range(NSTR):
                cps.append(pltpu.async_copy(
                    tab.at[ia_v.at[j]], rowsa.at[pl.ds(j * SUB, SUB)], sem))
            for cp in cps:
                cp.wait()
            pltpu.sync_copy(rowsa, outb.at[pl.ds(wid * PW + g * CH, CH)])
            return carry

        lax.fori_loop(0, NCH, body_hi, 0)

    return k(x1lo, x1hi, srci, dsti)


def _sc_scatter1(tpa, tpb, idx2, zrows, zcnt, ones):
    @functools.partial(
        pl.kernel,
        out_type=[jax.ShapeDtypeStruct((2 * N3P, TD), jnp.float32),
                  jax.ShapeDtypeStruct((2 * N3P, TD), jnp.float32),
                  jax.ShapeDtypeStruct((2 * N3P,), jnp.float32)],
        mesh=_sc_mesh(),
        compiler_params=_SC_PARAMS,
        scratch_types=[pltpu.VMEM((NSTR, SUB), jnp.int32),
                       pltpu.VMEM((CH, TD), jnp.float32),
                       pltpu.VMEM((SUB,), jnp.float32),
                       pltpu.VMEM_SHARED((N3P, TD), jnp.float32),
                       pltpu.VMEM_SHARED((N3P,), jnp.float32)],
    )
    def k(va_hbm, vb_hbm, i_hbm, zr_hbm, zc_hbm, on_hbm,
          parta, partb, cntp, idx_v, vals_v, ones_v, acc, cacc):
        cid = lax.axis_index("c")
        sid = lax.axis_index("s")
        wid = cid * (NW // 2) + sid
        pltpu.sync_copy(zr_hbm, acc.at[pl.ds(sid * STRIPE, STRIPE)])
        pltpu.sync_copy(zc_hbm, cacc.at[pl.ds(sid * STRIPE, STRIPE)])
        pltpu.sync_copy(on_hbm, ones_v)
        plsc.subcore_barrier()

        def body_a(g, carry):
            r0 = wid * (PW // SUB) + g * NSTR
            pltpu.sync_copy(i_hbm.at[pl.ds(r0, NSTR)], idx_v)
            pltpu.sync_copy(va_hbm.at[pl.ds(wid * PW + g * CH, CH)], vals_v)
            for j in range(NSTR):
                pltpu.sync_copy(vals_v.at[pl.ds(j * SUB, SUB)],
                                acc.at[idx_v.at[j]], add=True)
                pltpu.sync_copy(ones_v, cacc.at[idx_v.at[j]], add=True)
            return carry

        lax.fori_loop(0, NCH, body_a, 0)
        plsc.subcore_barrier()
        pltpu.sync_copy(acc.at[pl.ds(sid * STRIPE, STRIPE)],
                        parta.at[pl.ds(cid * N3P + sid * STRIPE, STRIPE)])
        pltpu.sync_copy(cacc.at[pl.ds(sid * STRIPE, STRIPE)],
                        cntp.at[pl.ds(cid * N3P + sid * STRIPE, STRIPE)])
        pltpu.sync_copy(zr_hbm, acc.at[pl.ds(sid * STRIPE, STRIPE)])
        plsc.subcore_barrier()

        def body_b(g, carry):
            r0 = wid * (PW // SUB) + g * NSTR
            pltpu.sync_copy(i_hbm.at[pl.ds(r0, NSTR)], idx_v)
            pltpu.sync_copy(vb_hbm.at[pl.ds(wid * PW + g * CH, CH)], vals_v)
            for j in range(NSTR):
                pltpu.sync_copy(vals_v.at[pl.ds(j * SUB, SUB)],
                                acc.at[idx_v.at[j]], add=True)
            return carry

        lax.fori_loop(0, NCH, body_b, 0)
        plsc.subcore_barrier()
        pltpu.sync_copy(acc.at[pl.ds(sid * STRIPE, STRIPE)],
                        partb.at[pl.ds(cid * N3P + sid * STRIPE, STRIPE)])

    return k(tpa, tpb, idx2, zrows, zcnt, ones)


def _sc_scatter2(tp2, idx2, zrows16):
    @functools.partial(
        pl.kernel,
        out_type=jax.ShapeDtypeStruct((2 * N3P, NS), jnp.float32),
        mesh=_sc_mesh(),
        compiler_params=_SC_PARAMS,
        scratch_types=[pltpu.VMEM((NSTR, SUB), jnp.int32),
                       pltpu.VMEM((CH, NS), jnp.float32),
                       pltpu.VMEM_SHARED((N3P, NS), jnp.float32)],
    )
    def k(v_hbm, i_hbm, zr_hbm, part, idx_v, vals_v, acc):
        cid = lax.axis_index("c")
        sid = lax.axis_index("s")
        wid = cid * (NW // 2) + sid
        pltpu.sync_copy(zr_hbm, acc.at[pl.ds(sid * STRIPE, STRIPE)])
        plsc.subcore_barrier()

        def body(g, carry):
            r0 = wid * (PW // SUB) + g * NSTR
            pltpu.sync_copy(i_hbm.at[pl.ds(r0, NSTR)], idx_v)
            pltpu.sync_copy(v_hbm.at[pl.ds(wid * PW + g * CH, CH)], vals_v)
            for j in range(NSTR):
                pltpu.sync_copy(vals_v.at[pl.ds(j * SUB, SUB)],
                                acc.at[idx_v.at[j]], add=True)
            return carry

        lax.fori_loop(0, NCH, body, 0)
        plsc.subcore_barrier()
        pltpu.sync_copy(acc.at[pl.ds(sid * STRIPE, STRIPE)],
                        part.at[pl.ds(cid * N3P + sid * STRIPE, STRIPE)])

    return k(tp2, idx2, zrows16)


def kernel(x_r0, pos_r0, edge_index_r0, batch_r0,
           x_r1, pos_r1, edge_index_r1, batch_r1,
           x_p, pos_p, edge_index_p, batch_p, params):
    p = params
    npad = N3P - N3
    epad = E3P - E3
    x_all = jnp.concatenate(
        [x_r0, x_r1, x_p, jnp.zeros((npad, NODE_FDIM), jnp.float32)], axis=0)
    pos_all = jnp.concatenate(
        [pos_r0, pos_r1, pos_p, jnp.zeros((npad, 3), jnp.float32)], axis=0)
    src_all = jnp.concatenate([edge_index_r0[0], edge_index_r1[0] + N,
                               edge_index_p[0] + 2 * N,
                               jnp.zeros((epad,), jnp.int32)])
    dst_all = jnp.concatenate([edge_index_r0[1], edge_index_r1[1] + N,
                               edge_index_p[1] + 2 * N,
                               jnp.full((epad,), N3, jnp.int32)])
    batf = jnp.concatenate(
        [batch_r0, batch_r1 + NG, batch_p + 2 * NG,
         jnp.full((npad,), -1, jnp.int32)]).astype(jnp.float32).reshape(N3P, 1)
    src2 = src_all.reshape(NIR, SUB)
    dst2 = dst_all.reshape(NIR, SUB)

    t1 = _node_mlp(x_all, pos_all, p)
    gsrc, gdst = _sc_gather1(t1, src2, dst2)
    tpa, tpb, ea, sh = _edge1(gsrc, gdst, p)
    zrows = jnp.zeros((STRIPE, TD), jnp.float32)
    zcnt = jnp.zeros((STRIPE,), jnp.float32)
    ones = jnp.ones((SUB,), jnp.float32)
    parta, partb, cntp = _sc_scatter1(tpa, tpb, dst2, zrows, zcnt, ones)
    cnt = (cntp[:N3P] + cntp[N3P:]).reshape(N3P, 1)
    x1lo, x1hi = _x1_stage(t1, parta[:N3P], parta[N3P:],
                           partb[:N3P], partb[N3P:], cnt)
    ga, gd, gb = _sc_gather2(x1lo, x1hi, src2, dst2)
    tp2 = _edge2(ga, gb, gd, ea, sh, p)
    zrows16 = jnp.zeros((STRIPE, NS), jnp.float32)
    part2 = _sc_scatter2(tp2, dst2, zrows16)
    s48 = _final_stage(x1lo, part2[:N3P], part2[N3P:], cnt, batf, p)[0]
    return (s48[2 * NG:3 * NG] - s48[0:NG] - s48[NG:2 * NG]).reshape(NG, 1)
